# baseline (speedup 1.0000x reference)
import jax
import jax.numpy as jnp
from jax import lax
from jax.experimental import pallas as pl
from jax.experimental.pallas import tpu as pltpu
from jax.experimental.pallas import tpu_sc as plsc

_B, _S, _D = 4, 2048, 1024
_NW = 32
_WRPW = _S // _NW
_CR = 32
_NC = _WRPW // _CR
_CE = _CR * _D
_NITEM = _NC * _B


def _sc_body(x_hbm, w_hbm, o_hbm, xb, wb, sem_x, sem_w, sem_o):
    wid = lax.axis_index("s") * 2 + lax.axis_index("c")
    w_base = wid * _WRPW * _D

    def item(k):
        c, b = divmod(k, _B)
        return c, b, b * _S * _D + w_base + c * _CE

    def start_x(k, slot):
        _, _, off = item(k)
        pltpu.make_async_copy(x_hbm.at[pl.ds(off, _CE)], xb.at[slot],
                              sem_x.at[slot]).start()

    pltpu.make_async_copy(w_hbm.at[pl.ds(w_base, _CE)], wb, sem_w).start()
    start_x(0, 0)

    for k in range(_NITEM):
        c, b, off = item(k)
        slot = k % 2
        if k + 1 < _NITEM:
            slot2 = (k + 1) % 2
            if k + 1 >= 2:
                pltpu.make_async_copy(xb.at[slot2], o_hbm.at[pl.ds(0, _CE)],
                                      sem_o.at[slot2]).wait()
            start_x(k + 1, slot2)
        pltpu.make_async_copy(x_hbm.at[pl.ds(0, _CE)], xb.at[slot],
                              sem_x.at[slot]).wait()
        if b == 0:
            pltpu.make_async_copy(w_hbm.at[pl.ds(0, _CE)], wb,
                                  sem_w).wait()

        xr = xb.at[slot]

        def add16(j, _):
            s = pl.ds(j * 16, 16)
            xr[s] = xr[s] + wb[s]
            return 0

        lax.fori_loop(0, _CE // 16, add16, 0, unroll=8)
        if b == _B - 1 and c + 1 < _NC:
            pltpu.make_async_copy(w_hbm.at[pl.ds(w_base + (c + 1) * _CE, _CE)],
                                  wb, sem_w).start()
        pltpu.make_async_copy(xb.at[slot], o_hbm.at[pl.ds(off, _CE)],
                              sem_o.at[slot]).start()

    for slot in (_NITEM % 2, (_NITEM + 1) % 2):
        pltpu.make_async_copy(xb.at[slot], o_hbm.at[pl.ds(0, _CE)],
                              sem_o.at[slot]).wait()


def kernel(x, weight):
    B, S, D = x.shape
    xf = x.reshape(B * S * D)
    wf = weight[:S].reshape(S * D)
    mesh = plsc.VectorSubcoreMesh(core_axis_name="c", subcore_axis_name="s")
    run = pl.kernel(
        _sc_body,
        mesh=mesh,
        out_type=jax.ShapeDtypeStruct((B * S * D,), x.dtype),
        scratch_types=[
            pltpu.VMEM((2, _CE), x.dtype),
            pltpu.VMEM((_CE,), x.dtype),
            pltpu.SemaphoreType.DMA((2,)),
            pltpu.SemaphoreType.DMA,
            pltpu.SemaphoreType.DMA((2,)),
        ],
    )
    out = run(xf, wf)
    return out.reshape(B, S, D)


# speedup vs baseline: 8.3374x; 8.3374x over previous
import jax
import jax.numpy as jnp
from jax.experimental import pallas as pl
from jax.experimental.pallas import tpu as pltpu

_SCHED = (
    (0, 0, 256),
    (0, 256, 1792),
    (1, 0, 2048),
    (2, 0, 2048),
    (3, 0, 1792),
    (3, 1792, 256),
)
_WPIECES = ((0, 256), (256, 1792))


def _body(x_hbm, w_hbm, o_hbm, xb, wb, ob, sem_x, sem_w, sem_o):
    N = len(_SCHED)

    def x_in(c, start=True):
        b, r, n = _SCHED[c]
        cp = pltpu.make_async_copy(
            x_hbm.at[b, pl.ds(r, n), :], xb.at[c % 2, pl.ds(0, n), :],
            sem_x.at[c % 2])
        cp.start() if start else cp.wait()

    def o_out(c, start=True):
        b, r, n = _SCHED[c]
        cp = pltpu.make_async_copy(
            ob.at[c % 2, pl.ds(0, n), :], o_hbm.at[b, pl.ds(r, n), :],
            sem_o.at[c % 2])
        cp.start() if start else cp.wait()

    def w_in(p, start=True):
        r, n = _WPIECES[p]
        cp = pltpu.make_async_copy(
            w_hbm.at[pl.ds(r, n), :], wb.at[pl.ds(r, n), :], sem_w.at[p])
        cp.start() if start else cp.wait()

    w_in(0)
    x_in(0)
    w_in(1)
    x_in(1)

    for c in range(N):
        b, r, n = _SCHED[c]
        slot = c % 2
        x_in(c, start=False)
        if c < len(_WPIECES):
            w_in(c, start=False)
        if c >= 2:
            o_out(c - 2, start=False)
        ob[slot, :n] = xb[slot, :n] + wb[r:r + n]
        o_out(c)
        if c + 2 < N:
            x_in(c + 2)

    o_out(N - 2, start=False)
    o_out(N - 1, start=False)


def kernel(x, weight):
    B, S, D = x.shape
    w = weight[:S]
    return pl.pallas_call(
        _body,
        in_specs=[
            pl.BlockSpec(memory_space=pl.ANY),
            pl.BlockSpec(memory_space=pl.ANY),
        ],
        out_specs=pl.BlockSpec(memory_space=pl.ANY),
        out_shape=jax.ShapeDtypeStruct((B, S, D), x.dtype),
        scratch_shapes=[
            pltpu.VMEM((2, S, D), x.dtype),
            pltpu.VMEM((S, D), x.dtype),
            pltpu.VMEM((2, S, D), x.dtype),
            pltpu.SemaphoreType.DMA((2,)),
            pltpu.SemaphoreType.DMA((2,)),
            pltpu.SemaphoreType.DMA((2,)),
        ],
        compiler_params=pltpu.CompilerParams(vmem_limit_bytes=56 * 1024 * 1024),
    )(x, w)
